# baseline (speedup 1.0000x reference)
import jax
import jax.numpy as jnp
from jax import lax
from jax.experimental import pallas as pl
from jax.experimental.pallas import tpu as pltpu

H = 512
W = 512
STUFF = 53
THING = 80
NUM_INST = 64
MSIZE = 100
BLK = 128
STRIP = 2 * BLK

_CH, _BY0, _BX0, _BH, _BW, _CY2, _CX2 = range(7)


def _sblk(s, n):
    return jnp.minimum(s[_BY0, n] // BLK, H // BLK - 2)


def _copy_map(c, s):
    return (jnp.minimum(c, STUFF - 1), 0, 0)


def _gather0_map(c, s):
    n = jnp.maximum(c - STUFF, 0)
    return (jnp.where(c < STUFF, STUFF, s[_CH, n]),
            jnp.where(c < STUFF, 0, _sblk(s, n)), 0)


def _gather1_map(c, s):
    n = jnp.maximum(c - STUFF, 0)
    return (jnp.where(c < STUFF, STUFF, s[_CH, n]),
            jnp.where(c < STUFF, 0, _sblk(s, n) + 1), 0)


def _mask_map(c, s):
    return (jnp.maximum(c - STUFF, 0), 0, 0)


def _out_map(c, s):
    return (c, 0, 0)


def _body(s, semc_ref, semg0_ref, semg1_ref, mask_ref, out_ref):
    c = pl.program_id(0)

    @pl.when(c < STUFF)
    def _():
        out_ref[...] = semc_ref[...]

    @pl.when(c >= STUFF)
    def _():
        n = c - STUFF
        by0 = s[_BY0, n]
        bx0 = s[_BX0, n]
        bh = s[_BH, n]
        bw = s[_BW, n]
        cy2 = s[_CY2, n]
        cx2 = s[_CX2, n]
        hbase = jnp.minimum(by0 // BLK, H // BLK - 2) * BLK
        by0f = by0.astype(jnp.float32)
        bx0f = bx0.astype(jnp.float32)
        bhf = bh.astype(jnp.float32)
        bwf = bw.astype(jnp.float32)

        out_ref[...] = jnp.zeros_like(out_ref)

        h = (lax.broadcasted_iota(jnp.int32, (STRIP, 128), 0) + hbase).astype(jnp.float32)
        m = lax.broadcasted_iota(jnp.int32, (STRIP, 128), 1).astype(jnp.float32)
        sy = (h - by0f + 0.5) * (MSIZE / bhf) - 0.5
        sy = jnp.clip(sy, 0.0, MSIZE - 1.0)
        yf = jnp.floor(sy)
        wy = sy - yf
        ay = (m == yf) * (1.0 - wy) + (m == jnp.minimum(yf + 1.0, MSIZE - 1.0)) * wy
        rowin = (h >= by0f) & (h <= by0f + bhf - 1.0)
        ay = jnp.where(rowin, ay, 0.0)

        k = lax.broadcasted_iota(jnp.int32, (128, W), 0).astype(jnp.float32)
        xx = lax.broadcasted_iota(jnp.int32, (128, W), 1).astype(jnp.float32)
        sx = (xx - bx0f + 0.5) * (MSIZE / bwf) - 0.5
        sx = jnp.clip(sx, 0.0, MSIZE - 1.0)
        xf = jnp.floor(sx)
        wx = sx - xf
        axt = (k == xf) * (1.0 - wx) + (k == jnp.minimum(xf + 1.0, MSIZE - 1.0)) * wx
        colin = (xx >= bx0f) & (xx <= bx0f + bwf - 1.0)
        axt = jnp.where(colin, axt, 0.0)

        t = jnp.dot(ay, mask_ref[0], precision=lax.Precision.DEFAULT,
                    preferred_element_type=jnp.float32)
        p = jnp.dot(t, axt, precision=lax.Precision.DEFAULT,
                    preferred_element_type=jnp.float32)

        hi = lax.broadcasted_iota(jnp.int32, (STRIP, W), 0) + hbase
        xi = lax.broadcasted_iota(jnp.int32, (STRIP, W), 1)
        cm = (hi >= by0) & (hi < cy2) & (xi >= bx0) & (xi < cx2)
        sel = jnp.concatenate([semg0_ref[0], semg1_ref[0]], axis=0)
        res = p + jnp.where(cm, sel, 0.0)

        out_ref[0, pl.ds(hbase, STRIP), :] = res


def _grid_spec():
    return pltpu.PrefetchScalarGridSpec(
        num_scalar_prefetch=1,
        grid=(STUFF + NUM_INST,),
        in_specs=[
            pl.BlockSpec((1, H, W), _copy_map),
            pl.BlockSpec((1, BLK, W), _gather0_map),
            pl.BlockSpec((1, BLK, W), _gather1_map),
            pl.BlockSpec((1, 128, 128), _mask_map),
        ],
        out_specs=pl.BlockSpec((1, H, W), _out_map),
    )


def _prep(sem_seg_logits, mask_logits, boxes, cls_idx):
    sem = sem_seg_logits.reshape(STUFF + THING, H, W)
    mask = mask_logits.reshape(NUM_INST, MSIZE, MSIZE)
    maskp = jnp.pad(mask, ((0, 0), (0, 128 - MSIZE), (0, 128 - MSIZE)))
    bx0 = boxes[:, 0].astype(jnp.int32)
    by0 = boxes[:, 1].astype(jnp.int32)
    bx1 = boxes[:, 2].astype(jnp.int32)
    by1 = boxes[:, 3].astype(jnp.int32)
    bw = bx1 - bx0 + 1
    bh = by1 - by0 + 1
    cx2 = jnp.round(boxes[:, 2]).astype(jnp.int32) + 1
    cy2 = jnp.round(boxes[:, 3]).astype(jnp.int32) + 1
    ch = STUFF + cls_idx.astype(jnp.int32)
    scal = jnp.stack([ch, by0, bx0, bh, bw, cy2, cx2,
                      jnp.zeros_like(ch)])
    return scal, sem, maskp


def kernel(sem_seg_logits, mask_logits, boxes, cls_idx):
    scal, sem, maskp = _prep(sem_seg_logits, mask_logits, boxes, cls_idx)
    out = pl.pallas_call(
        _body,
        grid_spec=_grid_spec(),
        out_shape=jax.ShapeDtypeStruct((STUFF + NUM_INST, H, W), jnp.float32),
        compiler_params=pltpu.CompilerParams(
            dimension_semantics=("arbitrary",)),
    )(scal, sem, sem, sem, maskp)
    return out.reshape(1, STUFF + NUM_INST, H, W)


# speedup vs baseline: 1.0060x; 1.0060x over previous
import jax
import jax.numpy as jnp
from jax import lax
from jax.experimental import pallas as pl
from jax.experimental.pallas import tpu as pltpu

H = 512
W = 512
STUFF = 53
THING = 80
NUM_INST = 64
MSIZE = 100
BLK = 128
STRIP = 2 * BLK

_CH, _BY0, _BX0, _BH, _BW, _CY2, _CX2 = range(7)


def _sblk(s, n):
    return jnp.minimum(s[_BY0, n] // BLK, H // BLK - 2)


def _copy_map(c, s):
    return (jnp.minimum(c, STUFF - 1), 0, 0)


def _gather0_map(c, s):
    n = jnp.maximum(c - STUFF, 0)
    return (jnp.where(c < STUFF, STUFF, s[_CH, n]),
            jnp.where(c < STUFF, 0, _sblk(s, n)), 0)


def _gather1_map(c, s):
    n = jnp.maximum(c - STUFF, 0)
    return (jnp.where(c < STUFF, STUFF, s[_CH, n]),
            jnp.where(c < STUFF, 0, _sblk(s, n) + 1), 0)


def _mask_map(c, s):
    return (jnp.maximum(c - STUFF, 0), 0, 0)


def _out_map(c, s):
    return (c, 0, 0)


def _body(s, semc_ref, semg0_ref, semg1_ref, mask_ref, out_ref):
    c = pl.program_id(0)

    @pl.when(c < STUFF)
    def _():
        out_ref[...] = semc_ref[...]

    @pl.when(c >= STUFF)
    def _():
        n = c - STUFF
        by0 = s[_BY0, n]
        bx0 = s[_BX0, n]
        bh = s[_BH, n]
        bw = s[_BW, n]
        cy2 = s[_CY2, n]
        cx2 = s[_CX2, n]
        hbase = jnp.minimum(by0 // BLK, H // BLK - 2) * BLK
        by0f = by0.astype(jnp.float32)
        bx0f = bx0.astype(jnp.float32)
        bhf = bh.astype(jnp.float32)
        bwf = bw.astype(jnp.float32)

        z = jnp.zeros((BLK, W), dtype=jnp.float32)
        zoff_a = jnp.where(by0 // BLK == 0, 2 * BLK, 0)
        zoff_b = jnp.where(by0 // BLK >= 2, BLK, 3 * BLK)
        out_ref[0, pl.ds(zoff_a, BLK), :] = z
        out_ref[0, pl.ds(zoff_b, BLK), :] = z

        h = (lax.broadcasted_iota(jnp.int32, (STRIP, 128), 0) + hbase).astype(jnp.float32)
        m = lax.broadcasted_iota(jnp.int32, (STRIP, 128), 1).astype(jnp.float32)
        sy = (h - by0f + 0.5) * (MSIZE / bhf) - 0.5
        sy = jnp.clip(sy, 0.0, MSIZE - 1.0)
        yf = jnp.floor(sy)
        wy = sy - yf
        ay = (m == yf) * (1.0 - wy) + (m == jnp.minimum(yf + 1.0, MSIZE - 1.0)) * wy
        rowin = (h >= by0f) & (h <= by0f + bhf - 1.0)
        ay = jnp.where(rowin, ay, 0.0)

        k = lax.broadcasted_iota(jnp.int32, (128, W), 0).astype(jnp.float32)
        xx = lax.broadcasted_iota(jnp.int32, (128, W), 1).astype(jnp.float32)
        sx = (xx - bx0f + 0.5) * (MSIZE / bwf) - 0.5
        sx = jnp.clip(sx, 0.0, MSIZE - 1.0)
        xf = jnp.floor(sx)
        wx = sx - xf
        axt = (k == xf) * (1.0 - wx) + (k == jnp.minimum(xf + 1.0, MSIZE - 1.0)) * wx
        colin = (xx >= bx0f) & (xx <= bx0f + bwf - 1.0)
        axt = jnp.where(colin, axt, 0.0)

        t = jnp.dot(ay, mask_ref[0], precision=lax.Precision.DEFAULT,
                    preferred_element_type=jnp.float32)
        p = jnp.dot(t, axt, precision=lax.Precision.DEFAULT,
                    preferred_element_type=jnp.float32)

        hi = lax.broadcasted_iota(jnp.int32, (STRIP, W), 0) + hbase
        xi = lax.broadcasted_iota(jnp.int32, (STRIP, W), 1)
        cm = (hi >= by0) & (hi < cy2) & (xi >= bx0) & (xi < cx2)
        sel = jnp.concatenate([semg0_ref[0], semg1_ref[0]], axis=0)
        res = p + jnp.where(cm, sel, 0.0)

        out_ref[0, pl.ds(hbase, STRIP), :] = res


def _grid_spec():
    return pltpu.PrefetchScalarGridSpec(
        num_scalar_prefetch=1,
        grid=(STUFF + NUM_INST,),
        in_specs=[
            pl.BlockSpec((1, H, W), _copy_map),
            pl.BlockSpec((1, BLK, W), _gather0_map),
            pl.BlockSpec((1, BLK, W), _gather1_map),
            pl.BlockSpec((1, 128, 128), _mask_map),
        ],
        out_specs=pl.BlockSpec((1, H, W), _out_map),
    )


def _prep(sem_seg_logits, mask_logits, boxes, cls_idx):
    sem = sem_seg_logits.reshape(STUFF + THING, H, W)
    mask = mask_logits.reshape(NUM_INST, MSIZE, MSIZE)
    maskp = jnp.pad(mask, ((0, 0), (0, 128 - MSIZE), (0, 128 - MSIZE)))
    bx0 = boxes[:, 0].astype(jnp.int32)
    by0 = boxes[:, 1].astype(jnp.int32)
    bx1 = boxes[:, 2].astype(jnp.int32)
    by1 = boxes[:, 3].astype(jnp.int32)
    bw = bx1 - bx0 + 1
    bh = by1 - by0 + 1
    cx2 = jnp.round(boxes[:, 2]).astype(jnp.int32) + 1
    cy2 = jnp.round(boxes[:, 3]).astype(jnp.int32) + 1
    ch = STUFF + cls_idx.astype(jnp.int32)
    scal = jnp.stack([ch, by0, bx0, bh, bw, cy2, cx2,
                      jnp.zeros_like(ch)])
    return scal, sem, maskp


def kernel(sem_seg_logits, mask_logits, boxes, cls_idx):
    scal, sem, maskp = _prep(sem_seg_logits, mask_logits, boxes, cls_idx)
    out = pl.pallas_call(
        _body,
        grid_spec=_grid_spec(),
        out_shape=jax.ShapeDtypeStruct((STUFF + NUM_INST, H, W), jnp.float32),
        compiler_params=pltpu.CompilerParams(
            dimension_semantics=("arbitrary",)),
    )(scal, sem, sem, sem, maskp)
    return out.reshape(1, STUFF + NUM_INST, H, W)
